# Initial kernel scaffold; baseline (speedup 1.0000x reference)
import jax
import jax.numpy as jnp
from jax.experimental import pallas as pl


def kernel(day, month, year, weekday, day_table, month_table, year_table, weekday_table):
    raise NotImplementedError("write your pallas kernel here")


# speedup vs baseline: 7.6316x; 7.6316x over previous
import functools

import jax
import jax.numpy as jnp
from jax import lax
from jax.experimental import pallas as pl
from jax.experimental.pallas import tpu as pltpu
from jax.experimental.pallas import tpu_sc as plsc

B = 16384
L = 50
SUB = 32
N = B * L
NW = 32
PER_W = N // NW
C = 128
N_CHUNKS = PER_W // C

PAD_ROWS = (32, 16, 104, 8)

_mesh = plsc.VectorSubcoreMesh(core_axis_name="c", subcore_axis_name="s")


@functools.partial(
    pl.kernel,
    out_type=jax.ShapeDtypeStruct((N, 4 * SUB), jnp.float32),
    mesh=_mesh,
    scratch_types=[
        pltpu.VMEM_SHARED((PAD_ROWS[0], 4 * SUB), jnp.float32),
        pltpu.VMEM_SHARED((PAD_ROWS[1], 4 * SUB), jnp.float32),
        pltpu.VMEM_SHARED((PAD_ROWS[2], 4 * SUB), jnp.float32),
        pltpu.VMEM_SHARED((PAD_ROWS[3], 4 * SUB), jnp.float32),
        pltpu.VMEM((max(PAD_ROWS), 4 * SUB), jnp.float32),
        pltpu.VMEM((C,), jnp.int32),
        pltpu.VMEM((C, 4 * SUB), jnp.float32),
        pltpu.SemaphoreType.DMA,
    ],
)
def _emb_kernel(day, month, year, weekday, dt, mt, yt, wt, out,
                dt_s, mt_s, yt_s, wt_s, stage_v, idx_v, rows_v, sem):
    wid = lax.axis_index("s") * 2 + lax.axis_index("c")
    base_w = wid * PER_W

    @pl.when(lax.axis_index("s") == 0)
    def _stage():
        for tab_hbm, tab_s, nrows in ((dt, dt_s, PAD_ROWS[0]),
                                      (mt, mt_s, PAD_ROWS[1]),
                                      (yt, yt_s, PAD_ROWS[2]),
                                      (wt, wt_s, PAD_ROWS[3])):
            pltpu.sync_copy(tab_hbm, stage_v.at[pl.ds(0, nrows), :])
            pltpu.sync_copy(stage_v.at[pl.ds(0, nrows), :], tab_s)

    plsc.subcore_barrier()

    def chunk_body(ci, carry):
        base = base_w + ci * C
        pltpu.sync_copy(day.at[pl.ds(base, C)], idx_v)
        pltpu.async_copy(dt_s.at[idx_v], rows_v, sem).wait()
        for idx_hbm, tab_s in ((month, mt_s), (year, yt_s), (weekday, wt_s)):
            pltpu.sync_copy(idx_hbm.at[pl.ds(base, C)], idx_v)
            pltpu.async_copy(tab_s.at[idx_v], rows_v, sem, add=True).wait()
        pltpu.sync_copy(rows_v, out.at[pl.ds(base, C), :])
        return carry

    lax.fori_loop(0, N_CHUNKS, chunk_body, 0)


def _pad_table(tab, rows8, j):
    out = jnp.zeros((rows8, 4 * SUB), tab.dtype)
    return lax.dynamic_update_slice(out, tab, (0, j * SUB))


def kernel(day, month, year, weekday,
           day_table, month_table, year_table, weekday_table):
    out = _emb_kernel(
        day.reshape(N), month.reshape(N), year.reshape(N), weekday.reshape(N),
        _pad_table(day_table, PAD_ROWS[0], 0),
        _pad_table(month_table, PAD_ROWS[1], 1),
        _pad_table(year_table, PAD_ROWS[2], 2),
        _pad_table(weekday_table, PAD_ROWS[3], 3))
    return out.reshape(B, L, 4 * SUB)


# speedup vs baseline: 8.7750x; 1.1498x over previous
import functools

import jax
import jax.numpy as jnp
from jax import lax
from jax.experimental import pallas as pl
from jax.experimental.pallas import tpu as pltpu
from jax.experimental.pallas import tpu_sc as plsc

B = 16384
L = 50
SUB = 32
N = B * L
NW = 32
PER_W = N // NW
C = 128
N_CHUNKS = PER_W // C

DM_ROWS = 32 * 13
YW_ROWS = 100 * 7 + 4
STAGE_R = 32

_mesh = plsc.VectorSubcoreMesh(core_axis_name="c", subcore_axis_name="s")


@functools.partial(
    pl.kernel,
    out_type=jax.ShapeDtypeStruct((N, 4 * SUB), jnp.float32),
    mesh=_mesh,
    scratch_types=[
        pltpu.VMEM_SHARED((DM_ROWS, 4 * SUB), jnp.float32),
        pltpu.VMEM_SHARED((YW_ROWS, 4 * SUB), jnp.float32),
        pltpu.VMEM((STAGE_R, 4 * SUB), jnp.float32),
        pltpu.VMEM((C,), jnp.int32),
        pltpu.VMEM((C,), jnp.int32),
        pltpu.VMEM((C,), jnp.int32),
        pltpu.VMEM((C,), jnp.int32),
        pltpu.VMEM((C, 4 * SUB), jnp.float32),
        pltpu.SemaphoreType.DMA,
    ],
)
def _emb_kernel(day, month, year, weekday, dmt, ywt, out,
                dm_s, yw_s, stage_v, dmi_v, mi_v, ywi_v, wi_v, rows_v, sem):
    wid = lax.axis_index("s") * 2 + lax.axis_index("c")
    base_w = wid * PER_W

    @pl.when(lax.axis_index("s") == 0)
    def _stage():
        for tab_hbm, tab_s, nrows in ((dmt, dm_s, DM_ROWS),
                                      (ywt, yw_s, YW_ROWS)):
            for r0 in range(0, nrows, STAGE_R):
                pltpu.sync_copy(tab_hbm.at[pl.ds(r0, STAGE_R), :], stage_v)
                pltpu.sync_copy(stage_v, tab_s.at[pl.ds(r0, STAGE_R), :])

    plsc.subcore_barrier()

    def chunk_body(ci, carry):
        base = base_w + ci * C
        pltpu.sync_copy(day.at[pl.ds(base, C)], dmi_v)
        pltpu.sync_copy(month.at[pl.ds(base, C)], mi_v)
        pltpu.sync_copy(year.at[pl.ds(base, C)], ywi_v)
        pltpu.sync_copy(weekday.at[pl.ds(base, C)], wi_v)
        for g in range(C // 16):
            s = pl.ds(g * 16, 16)
            dmi_v[s] = dmi_v[s] * 13 + mi_v[s]
            ywi_v[s] = ywi_v[s] * 7 + wi_v[s]
        pltpu.async_copy(dm_s.at[dmi_v], rows_v, sem).wait()
        pltpu.async_copy(yw_s.at[ywi_v], rows_v, sem, add=True).wait()
        pltpu.sync_copy(rows_v, out.at[pl.ds(base, C), :])
        return carry

    lax.fori_loop(0, N_CHUNKS, chunk_body, 0)


def _build_pair_tables(day_table, month_table, year_table, weekday_table):
    dm = jnp.concatenate([jnp.repeat(day_table, 13, axis=0),
                          jnp.tile(month_table, (32, 1))], axis=1)
    dm = jnp.pad(dm, ((0, 0), (0, 2 * SUB)))
    yw = jnp.concatenate([jnp.repeat(year_table, 7, axis=0),
                          jnp.tile(weekday_table, (100, 1))], axis=1)
    yw = jnp.pad(yw, ((0, YW_ROWS - 700), (2 * SUB, 0)))
    return dm, yw


def kernel(day, month, year, weekday,
           day_table, month_table, year_table, weekday_table):
    dm, yw = _build_pair_tables(day_table, month_table,
                                year_table, weekday_table)
    out = _emb_kernel(
        day.reshape(N), month.reshape(N), year.reshape(N), weekday.reshape(N),
        dm, yw)
    return out.reshape(B, L, 4 * SUB)


# speedup vs baseline: 12.1300x; 1.3823x over previous
import functools

import jax
import jax.numpy as jnp
from jax import lax
from jax.experimental import pallas as pl
from jax.experimental.pallas import tpu as pltpu
from jax.experimental.pallas import tpu_sc as plsc

B = 16384
L = 50
SUB = 32
N = B * L
NW = 32
PER_W = N // NW
C = 128
N_CHUNKS = PER_W // C
N_ITERS = N_CHUNKS // 2

DM_ROWS = 32 * 13
YW_ROWS = 100 * 7 + 4
STAGE_R = 32

_mesh = plsc.VectorSubcoreMesh(core_axis_name="c", subcore_axis_name="s")


@functools.partial(
    pl.kernel,
    out_type=jax.ShapeDtypeStruct((N, 4 * SUB), jnp.float32),
    mesh=_mesh,
    scratch_types=[
        pltpu.VMEM_SHARED((DM_ROWS, 4 * SUB), jnp.float32),
        pltpu.VMEM_SHARED((YW_ROWS, 4 * SUB), jnp.float32),
        pltpu.VMEM((STAGE_R, 4 * SUB), jnp.float32),
        pltpu.VMEM((C,), jnp.int32), pltpu.VMEM((C,), jnp.int32),
        pltpu.VMEM((C,), jnp.int32), pltpu.VMEM((C,), jnp.int32),
        pltpu.VMEM((C,), jnp.int32), pltpu.VMEM((C,), jnp.int32),
        pltpu.VMEM((C,), jnp.int32), pltpu.VMEM((C,), jnp.int32),
        pltpu.VMEM((C, 4 * SUB), jnp.float32),
        pltpu.VMEM((C, 4 * SUB), jnp.float32),
        pltpu.SemaphoreType.DMA, pltpu.SemaphoreType.DMA,
        pltpu.SemaphoreType.DMA,
        pltpu.SemaphoreType.DMA, pltpu.SemaphoreType.DMA,
    ],
)
def _emb_kernel(day, month, year, weekday, dmt, ywt, out,
                dm_s, yw_s, stage_v,
                di_a, mi_a, yi_a, wi_a, di_b, mi_b, yi_b, wi_b,
                rows_a, rows_b,
                sem_ia, sem_ib, sem_g, sem_oa, sem_ob):
    wid = lax.axis_index("s") * 2 + lax.axis_index("c")
    base_w = wid * PER_W

    idx_bufs = ((di_a, mi_a, yi_a, wi_a), (di_b, mi_b, yi_b, wi_b))
    idx_sems = (sem_ia, sem_ib)
    rows_bufs = (rows_a, rows_b)
    out_sems = (sem_oa, sem_ob)

    @pl.when(lax.axis_index("s") == 0)
    def _stage():
        for tab_hbm, tab_s, nrows in ((dmt, dm_s, DM_ROWS),
                                      (ywt, yw_s, YW_ROWS)):
            for r0 in range(0, nrows, STAGE_R):
                pltpu.sync_copy(tab_hbm.at[pl.ds(r0, STAGE_R), :], stage_v)
                pltpu.sync_copy(stage_v, tab_s.at[pl.ds(r0, STAGE_R), :])

    plsc.subcore_barrier()

    def fire_idx(b, base):
        di, mi, yi, wi = idx_bufs[b]
        sem = idx_sems[b]
        pltpu.async_copy(day.at[pl.ds(base, C)], di, sem)
        pltpu.async_copy(month.at[pl.ds(base, C)], mi, sem)
        pltpu.async_copy(year.at[pl.ds(base, C)], yi, sem)
        pltpu.async_copy(weekday.at[pl.ds(base, C)], wi, sem)

    def drain_idx(b, base):
        di, mi, yi, wi = idx_bufs[b]
        sem = idx_sems[b]
        pltpu.make_async_copy(day.at[pl.ds(base, C)], di, sem).wait()
        pltpu.make_async_copy(month.at[pl.ds(base, C)], mi, sem).wait()
        pltpu.make_async_copy(year.at[pl.ds(base, C)], yi, sem).wait()
        pltpu.make_async_copy(weekday.at[pl.ds(base, C)], wi, sem).wait()

    fire_idx(0, base_w)

    def iter_body(i, carry):
        for b in (0, 1):
            ci = i * 2 + b
            base = base_w + ci * C
            if b == 0:
                fire_idx(1, base + C)
            else:
                @pl.when(ci + 1 < N_CHUNKS)
                def _prefetch():
                    fire_idx(0, base + C)
            drain_idx(b, base)
            di, mi, yi, wi = idx_bufs[b]
            for g in range(C // 16):
                s = pl.ds(g * 16, 16)
                di[s] = di[s] * 13 + mi[s]
                yi[s] = yi[s] * 7 + wi[s]
            rows_v = rows_bufs[b]
            @pl.when(i >= 1)
            def _wait_prev_out():
                pltpu.make_async_copy(
                    rows_v, out.at[pl.ds(base_w, C), :], out_sems[b]).wait()
            pltpu.async_copy(dm_s.at[di], rows_v, sem_g).wait()
            pltpu.async_copy(yw_s.at[yi], rows_v, sem_g, add=True).wait()
            pltpu.async_copy(rows_v, out.at[pl.ds(base, C), :], out_sems[b])
        return carry

    lax.fori_loop(0, N_ITERS, iter_body, 0)

    for b in (0, 1):
        pltpu.make_async_copy(
            rows_bufs[b], out.at[pl.ds(base_w, C), :], out_sems[b]).wait()


def _build_pair_tables(day_table, month_table, year_table, weekday_table):
    dm = jnp.concatenate([jnp.repeat(day_table, 13, axis=0),
                          jnp.tile(month_table, (32, 1))], axis=1)
    dm = jnp.pad(dm, ((0, 0), (0, 2 * SUB)))
    yw = jnp.concatenate([jnp.repeat(year_table, 7, axis=0),
                          jnp.tile(weekday_table, (100, 1))], axis=1)
    yw = jnp.pad(yw, ((0, YW_ROWS - 700), (2 * SUB, 0)))
    return dm, yw


def kernel(day, month, year, weekday,
           day_table, month_table, year_table, weekday_table):
    dm, yw = _build_pair_tables(day_table, month_table,
                                year_table, weekday_table)
    out = _emb_kernel(
        day.reshape(N), month.reshape(N), year.reshape(N), weekday.reshape(N),
        dm, yw)
    return out.reshape(B, L, 4 * SUB)
